# Initial kernel scaffold; baseline (speedup 1.0000x reference)
import jax
import jax.numpy as jnp
from jax.experimental import pallas as pl


def kernel(node_fts, automic_num, edge_fts, edge_index, F_norm_edge, F_dig, node_deg_vec, node_deg_mat, lap_mat, k_eig_val, k_eig_vec, num_nodes, norm_n, batch_idx, params):
    raise NotImplementedError("write your pallas kernel here")


# speedup vs baseline: 1.0734x; 1.0734x over previous
import functools

import jax
import jax.numpy as jnp
import numpy as np
from jax.experimental import pallas as pl
from jax.experimental.pallas import tpu as pltpu

N = 10000
E = 160000
HID = 128
EMB = 16
H = HID + EMB
NG = 128
AVG_D_LOG = 2.8332

ROW_BLK = 1000


def _readout_body(x_ref, bidx_ref, wr1_ref, br1_ref, wr2_ref, br2_ref,
                  wlast_ref, blast_ref, out_ref, acc_ref):
    i = pl.program_id(0)
    nsteps = pl.num_programs(0)

    @pl.when(i == 0)
    def _init():
        acc_ref[...] = jnp.zeros_like(acc_ref)

    x = x_ref[...]
    out = x @ wlast_ref[...] + blast_ref[...]
    bidx = bidx_ref[0]
    gids = jax.lax.broadcasted_iota(jnp.int32, (NG, ROW_BLK), 0)
    onehot = (gids == bidx).astype(jnp.float32)
    ones_col = jnp.ones((ROW_BLK, 1), jnp.float32)
    xa = jnp.concatenate([out, ones_col], axis=1)
    acc_ref[...] += jnp.dot(onehot, xa, preferred_element_type=jnp.float32)

    @pl.when(i == nsteps - 1)
    def _fini():
        acc = acc_ref[...]
        cnt = jnp.maximum(acc[:, H:H + 1], 1.0)
        g = acc[:, :H] / cnt
        r1 = jnp.maximum(g @ wr1_ref[...] + br1_ref[...], 0.0)
        r = jnp.sum(r1 * wr2_ref[...], axis=1, keepdims=True) + br2_ref[0, 0]
        out_ref[...] = jnp.broadcast_to(r, (NG, NG))


def _readout(x, batch_idx, params):
    bidx3 = batch_idx.astype(jnp.int32).reshape(N // ROW_BLK, 1, ROW_BLK)
    grid = (N // ROW_BLK,)
    res = pl.pallas_call(
        _readout_body,
        grid=grid,
        in_specs=[
            pl.BlockSpec((ROW_BLK, H), lambda i: (i, 0)),
            pl.BlockSpec((1, 1, ROW_BLK), lambda i: (i, 0, 0)),
            pl.BlockSpec((H, H // 2), lambda i: (0, 0)),
            pl.BlockSpec((1, H // 2), lambda i: (0, 0)),
            pl.BlockSpec((1, H // 2), lambda i: (0, 0)),
            pl.BlockSpec((1, 1), lambda i: (0, 0)),
            pl.BlockSpec((H, H), lambda i: (0, 0)),
            pl.BlockSpec((1, H), lambda i: (0, 0)),
        ],
        out_specs=pl.BlockSpec((NG, NG), lambda i: (0, 0)),
        out_shape=jax.ShapeDtypeStruct((NG, NG), jnp.float32),
        scratch_shapes=[pltpu.VMEM((NG, H + 1), jnp.float32)],
    )(x, bidx3, params['W_r1'], params['b_r1'].reshape(1, -1),
      params['W_r2'].reshape(1, -1), params['b_r2'].reshape(1, 1),
      params['W_last'], params['b_last'].reshape(1, -1))
    return res[:, 0]


def kernel(node_fts, automic_num, edge_fts, edge_index, F_norm_edge, F_dig,
           node_deg_vec, node_deg_mat, lap_mat, k_eig_val, k_eig_vec,
           num_nodes, norm_n, batch_idx, params):
    e = edge_fts @ params['W_edge'] + params['b_edge']
    a = params['emb'][automic_num]
    x = node_fts @ params['W_node'] + params['b_node']
    x = jnp.concatenate([x, a], axis=1)
    x = x @ params['W_first'] + params['b_first']
    src = edge_index[0]
    dst = edge_index[1]
    deg = jax.ops.segment_sum(jnp.ones((E,), dtype=x.dtype), dst, num_segments=N)
    degc = jnp.maximum(deg, 1.0)
    d_log = jnp.log(deg + 1.0)
    s_amp = (d_log / AVG_D_LOG)[:, None]
    s_att = jnp.where(deg > 0, AVG_D_LOG / jnp.maximum(d_log, 1e-6), 1.0)[:, None]
    for lp in params['layers']:
        m = jnp.concatenate([x[src], x[dst], e], axis=1)
        m = jax.nn.relu(m @ lp['W_pre'] + lp['b_pre'])
        s = jax.ops.segment_sum(m, dst, num_segments=N)
        mean = s / degc[:, None]
        mx = jax.ops.segment_max(m, dst, num_segments=N)
        mx = jnp.where(deg[:, None] > 0, mx, 0.0)
        mn = -jax.ops.segment_max(-m, dst, num_segments=N)
        mn = jnp.where(deg[:, None] > 0, mn, 0.0)
        sq = jax.ops.segment_sum(m * m, dst, num_segments=N) / degc[:, None]
        std = jnp.sqrt(jax.nn.relu(sq - mean * mean) + 1e-5)
        agg = jnp.concatenate([mean, mx, mn, std], axis=1)
        agg = jnp.concatenate([agg, agg * s_amp, agg * s_att], axis=1)
        h = jnp.concatenate([x, agg], axis=1) @ lp['W_post'] + lp['b_post']
        h = h * norm_n
        mu = jnp.mean(h, axis=0)
        var = jnp.var(h, axis=0)
        h = (h - mu) / jnp.sqrt(var + 1e-5) * lp['gamma'] + lp['beta']
        h = jax.nn.relu(h)
        x = x + h
    return _readout(x, batch_idx, params)


# speedup vs baseline: 2.4475x; 2.2802x over previous
import functools

import jax
import jax.numpy as jnp
import numpy as np
from jax import lax
from jax.experimental import pallas as pl
from jax.experimental.pallas import tpu as pltpu
from jax.experimental.pallas import tpu_sc as plsc

N = 10000
E = 160000
HID = 128
H = 144
NG = 128
AVG_D_LOG = 2.8332

NC, NS = 2, 16
NW = NC * NS
RA = 64
NSW = 5
NSEG = NW * NSW
NPT = RA * NSW
N_PAD = NW * NPT
EW = 5120
E_PAD = NW * EW
C = 64
PC = 128
FT = 256
DCOL = 144
BIG = 1e30

ROW_BLK = 1000

_mesh = plsc.VectorSubcoreMesh(core_axis_name="c", subcore_axis_name="s")


def _wid():
    return lax.axis_index("s") * NC + lax.axis_index("c")


@functools.partial(
    pl.kernel, mesh=_mesh,
    out_type=jax.ShapeDtypeStruct((E_PAD, PC), jnp.float32),
    scratch_types=[
        pltpu.VMEM((PC,), jnp.int32),
        pltpu.VMEM((PC, PC), jnp.float32),
        pltpu.SemaphoreType.DMA,
    ],
)
def _permute_k(efts, permp, out, idx_v, rows_v, sem):
    base = _wid() * EW

    def body(ci, cr):
        off = base + ci * PC
        pltpu.sync_copy(permp.at[pl.ds(off, PC)], idx_v)
        pltpu.async_copy(efts.at[idx_v], rows_v, sem).wait()
        pltpu.sync_copy(rows_v, out.at[pl.ds(off, PC)])
        return cr

    lax.fori_loop(0, EW // PC, body, 0)


def _edge_body(t1, t2, e3, srcp, dstp, ebounds,
               s_o, q_o, x_o, n_o,
               bounds_v, sidx, didx, dscal, srows, drows, erows,
               acc_s, acc_q, acc_x, acc_n, sem):
    w = _wid()
    pltpu.sync_copy(ebounds, bounds_v.at[pl.ds(0, 168)])
    zero16 = jnp.zeros((16,), jnp.float32)
    big16 = jnp.full((16,), BIG, jnp.float32)
    one16 = jnp.ones((16,), jnp.float32)

    for r in range(NSW):
        seg = NSW * w + r
        n0 = seg * RA
        bv = bounds_v[pl.ds(seg, 16)]
        e_lo = bv[0]
        e_hi = bv[1]
        eb0 = (e_lo // 8) * 8
        nch = (e_hi - eb0 + (C - 1)) // C

        def initb(rr, cr):
            for v in range(FT // 16):
                cs = pl.ds(v * 16, 16)
                acc_s[rr, cs] = zero16
                acc_q[rr, cs] = zero16
                acc_x[rr, cs] = zero16
                acc_n[rr, cs] = big16
            return cr

        lax.fori_loop(0, RA, initb, 0)

        def chunk(ci, cr):
            eb = eb0 + ci * C
            pltpu.sync_copy(srcp.at[pl.ds(eb, C)], sidx)
            pltpu.sync_copy(dstp.at[pl.ds(eb, C)], didx)
            pltpu.sync_copy(dstp.at[pl.ds(eb, C)], dscal.at[pl.ds(0, C)])
            pltpu.async_copy(t1.at[sidx], srows, sem).wait()
            pltpu.async_copy(t2.at[didx], drows, sem).wait()
            pltpu.sync_copy(e3.at[pl.ds(eb, C)], erows)
            j_lo = jnp.maximum(e_lo - eb, 0)
            j_hi = jnp.minimum(C, e_hi - eb)

            def ebody(j, ecr):
                ld = dscal[pl.ds(j, 16)][0] - n0
                for v in range(H // 16):
                    cs = pl.ds(v * 16, 16)
                    m = jnp.maximum(
                        srows[j, cs] + drows[j, cs] + erows[j, cs], 0.0)
                    acc_s[ld, cs] += m
                    acc_q[ld, cs] += m * m
                    acc_x[ld, cs] = jnp.maximum(acc_x[ld, cs], m)
                    acc_n[ld, cs] = jnp.minimum(acc_n[ld, cs], m)
                acc_s[ld, pl.ds(DCOL, 16)] += one16
                return ecr

            lax.fori_loop(j_lo, j_hi, ebody, 0)
            return cr

        lax.fori_loop(0, nch, chunk, 0)

        rows = pl.ds(n0, RA)
        pltpu.sync_copy(acc_s, s_o.at[rows])
        pltpu.sync_copy(acc_q, q_o.at[rows])
        pltpu.sync_copy(acc_x, x_o.at[rows])
        pltpu.sync_copy(acc_n, n_o.at[rows])


_no = jax.ShapeDtypeStruct((N_PAD, FT), jnp.float32)
_edge_k = functools.partial(
    pl.kernel, mesh=_mesh,
    out_type=[_no, _no, _no, _no],
    scratch_types=[
        pltpu.VMEM((176,), jnp.int32),
        pltpu.VMEM((C,), jnp.int32),
        pltpu.VMEM((C,), jnp.int32),
        pltpu.VMEM((C + 16,), jnp.int32),
        pltpu.VMEM((C, FT), jnp.float32),
        pltpu.VMEM((C, FT), jnp.float32),
        pltpu.VMEM((C, H), jnp.float32),
        pltpu.VMEM((RA, FT), jnp.float32),
        pltpu.VMEM((RA, FT), jnp.float32),
        pltpu.VMEM((RA, FT), jnp.float32),
        pltpu.VMEM((RA, FT), jnp.float32),
        pltpu.SemaphoreType.DMA,
    ],
)(_edge_body)


EBLK = 2048


def _elin_body(ef_ref, w_ref, b_ref, o1, o2, o3):
    e3 = jnp.dot(ef_ref[...][:, :16], w_ref[...],
                 preferred_element_type=jnp.float32) + b_ref[...]
    for l, o in enumerate((o1, o2, o3)):
        o[...] = e3[:, l * H:(l + 1) * H]


def _edge_linear(efts_sp, wc, bc):
    e_spec = pl.BlockSpec((EBLK, H), lambda i: (i, 0))
    esh = jax.ShapeDtypeStruct((E_PAD, H), jnp.float32)
    return pl.pallas_call(
        _elin_body,
        grid=(E_PAD // EBLK,),
        in_specs=[
            pl.BlockSpec((EBLK, PC), lambda i: (i, 0)),
            pl.BlockSpec((16, 3 * H), lambda i: (0, 0)),
            pl.BlockSpec((1, 3 * H), lambda i: (0, 0)),
        ],
        out_specs=[e_spec, e_spec, e_spec],
        out_shape=[esh, esh, esh],
    )(efts_sp, wc, bc)


def _pad_table(t, nrows):
    return jnp.concatenate(
        [t, jnp.zeros((nrows, FT - H), jnp.float32)], axis=1)


NBLK = 2000


def _enc_body(nf_ref, af_ref, wn_ref, bn_ref, emb_ref, wf_ref, bf_ref,
              w1_ref, w2_ref, x0_ref, t1_ref, t2_ref):
    xw = jnp.dot(nf_ref[...], wn_ref[...],
                 preferred_element_type=jnp.float32) + bn_ref[...]
    ids = lax.broadcasted_iota(jnp.int32, (NBLK, 16), 1).astype(jnp.float32)
    oh = (ids == af_ref[...]).astype(jnp.float32)
    a = jnp.dot(oh, emb_ref[...], preferred_element_type=jnp.float32)
    x0 = jnp.dot(jnp.concatenate([xw, a], axis=1), wf_ref[...],
                 preferred_element_type=jnp.float32) + bf_ref[...]
    x0_ref[...] = x0
    t1 = jnp.dot(x0, w1_ref[...], preferred_element_type=jnp.float32)
    t2 = jnp.dot(x0, w2_ref[...], preferred_element_type=jnp.float32)
    t1_ref[...] = _pad_table(t1, NBLK)
    t2_ref[...] = _pad_table(t2, NBLK)


def _encoder(node_fts, automic_f, wn, bn, emb_pad, wf, bf, w1, w2):
    full = lambda r, c: pl.BlockSpec((r, c), lambda i: (0, 0))
    row = lambda c: pl.BlockSpec((NBLK, c), lambda i: (i, 0))
    return pl.pallas_call(
        _enc_body,
        grid=(N // NBLK,),
        in_specs=[
            row(HID), row(1), full(HID, HID), full(1, HID), full(16, 16),
            full(H, H), full(1, H), full(H, H), full(H, H),
        ],
        out_specs=[row(H), row(FT), row(FT)],
        out_shape=[
            jax.ShapeDtypeStruct((N, H), jnp.float32),
            jax.ShapeDtypeStruct((N, FT), jnp.float32),
            jax.ShapeDtypeStruct((N, FT), jnp.float32),
        ],
    )(node_fts, automic_f, wn, bn, emb_pad, wf, bf, w1, w2)


def _post1_body(s_ref, q_ref, x4_ref, n4_ref, x_ref, nrm_ref, wp_ref, bp_ref,
                h_ref, st_ref, acc_ref):
    i = pl.program_id(0)
    nsteps = pl.num_programs(0)

    @pl.when(i == 0)
    def _init():
        acc_ref[...] = jnp.zeros_like(acc_ref)

    sfull = s_ref[...]
    s = sfull[:, :H]
    deg = sfull[:, DCOL:DCOL + 1]
    sq = q_ref[...][:, :H]
    mx = x4_ref[...][:, :H]
    mn = n4_ref[...][:, :H]
    degc = jnp.maximum(deg, 1.0)
    dlog = jnp.log(deg + 1.0)
    s_amp = dlog / AVG_D_LOG
    s_att = jnp.where(deg > 0, AVG_D_LOG / jnp.maximum(dlog, 1e-6), 1.0)
    mn = jnp.where(deg > 0, mn, 0.0)
    mean = s / degc
    sqm = sq / degc
    std = jnp.sqrt(jnp.maximum(sqm - mean * mean, 0.0) + 1e-5)
    agg = jnp.concatenate([mean, mx, mn, std], axis=1)
    cat = jnp.concatenate([x_ref[...], agg, agg * s_amp, agg * s_att], axis=1)
    h = jnp.dot(cat, wp_ref[...], preferred_element_type=jnp.float32) + bp_ref[...]
    h = h * nrm_ref[...]
    h_ref[...] = h
    acc_ref[0:1, :] += jnp.sum(h, axis=0, keepdims=True)
    acc_ref[1:2, :] += jnp.sum(h * h, axis=0, keepdims=True)

    @pl.when(i == nsteps - 1)
    def _fini():
        st_ref[...] = acc_ref[...]


def _post1(s, q, x4, n4, x, nrm, wp, bp):
    t_spec = pl.BlockSpec((ROW_BLK, FT), lambda i: (i, 0))
    return pl.pallas_call(
        _post1_body,
        grid=(N // ROW_BLK,),
        in_specs=[
            t_spec, t_spec, t_spec, t_spec,
            pl.BlockSpec((ROW_BLK, H), lambda i: (i, 0)),
            pl.BlockSpec((ROW_BLK, 1), lambda i: (i, 0)),
            pl.BlockSpec((13 * H, H), lambda i: (0, 0)),
            pl.BlockSpec((1, H), lambda i: (0, 0)),
        ],
        out_specs=[
            pl.BlockSpec((ROW_BLK, H), lambda i: (i, 0)),
            pl.BlockSpec((8, H), lambda i: (0, 0)),
        ],
        out_shape=[
            jax.ShapeDtypeStruct((N, H), jnp.float32),
            jax.ShapeDtypeStruct((8, H), jnp.float32),
        ],
        scratch_shapes=[pltpu.VMEM((8, H), jnp.float32)],
    )(s, q, x4, n4, x, nrm, wp, bp)


def _make_post2(has_next):
    def body(*refs):
        if has_next:
            (h_ref, st_ref, x_ref, g_ref, b_ref, w1_ref, w2_ref,
             xn_ref, t1_ref, t2_ref) = refs
        else:
            h_ref, st_ref, x_ref, g_ref, b_ref, xn_ref = refs
        mu = st_ref[0:1, :] / N
        var = st_ref[1:2, :] / N - mu * mu
        rstd = lax.rsqrt(var + 1e-5)
        hn = (h_ref[...] - mu) * rstd * g_ref[...] + b_ref[...]
        xn = x_ref[...] + jnp.maximum(hn, 0.0)
        xn_ref[...] = xn
        if has_next:
            t1 = jnp.dot(xn, w1_ref[...], preferred_element_type=jnp.float32)
            t2 = jnp.dot(xn, w2_ref[...], preferred_element_type=jnp.float32)
            t1_ref[...] = _pad_table(t1, ROW_BLK)
            t2_ref[...] = _pad_table(t2, ROW_BLK)

    def run(h, stats, x, gamma, beta, w1=None, w2=None):
        row = lambda c: pl.BlockSpec((ROW_BLK, c), lambda i: (i, 0))
        in_specs = [
            row(H),
            pl.BlockSpec((8, H), lambda i: (0, 0)),
            row(H),
            pl.BlockSpec((1, H), lambda i: (0, 0)),
            pl.BlockSpec((1, H), lambda i: (0, 0)),
        ]
        args = [h, stats, x, gamma, beta]
        out_specs = [row(H)]
        out_shape = [jax.ShapeDtypeStruct((N, H), jnp.float32)]
        if has_next:
            in_specs += [pl.BlockSpec((H, H), lambda i: (0, 0))] * 2
            args += [w1, w2]
            out_specs += [row(FT)] * 2
            out_shape += [jax.ShapeDtypeStruct((N, FT), jnp.float32)] * 2
        return pl.pallas_call(
            body, grid=(N // ROW_BLK,), in_specs=in_specs,
            out_specs=out_specs, out_shape=out_shape)(*args)

    return run


_post2_next = _make_post2(True)
_post2_last = _make_post2(False)


def _readout_body(x_ref, bidx_ref, wr1_ref, br1_ref, wr2_ref, br2_ref,
                  wlast_ref, blast_ref, out_ref, acc_ref):
    i = pl.program_id(0)
    nsteps = pl.num_programs(0)

    @pl.when(i == 0)
    def _init():
        acc_ref[...] = jnp.zeros_like(acc_ref)

    x = x_ref[...]
    out = jnp.dot(x, wlast_ref[...],
                  preferred_element_type=jnp.float32) + blast_ref[...]
    bidx = bidx_ref[0]
    gids = lax.broadcasted_iota(jnp.int32, (NG, ROW_BLK), 0)
    onehot = (gids == bidx).astype(jnp.float32)
    ones_col = jnp.ones((ROW_BLK, 1), jnp.float32)
    xa = jnp.concatenate([out, ones_col], axis=1)
    acc_ref[...] += jnp.dot(onehot, xa, preferred_element_type=jnp.float32)

    @pl.when(i == nsteps - 1)
    def _fini():
        acc = acc_ref[...]
        cnt = jnp.maximum(acc[:, H:H + 1], 1.0)
        g = acc[:, :H] / cnt
        r1 = jnp.maximum(
            jnp.dot(g, wr1_ref[...], preferred_element_type=jnp.float32)
            + br1_ref[...], 0.0)
        r = jnp.sum(r1 * wr2_ref[...], axis=1, keepdims=True) + br2_ref[0, 0]
        out_ref[...] = jnp.broadcast_to(r, (NG, NG))


def _readout(x, batch_idx, params):
    bidx3 = batch_idx.astype(jnp.int32).reshape(N // ROW_BLK, 1, ROW_BLK)
    res = pl.pallas_call(
        _readout_body,
        grid=(N // ROW_BLK,),
        in_specs=[
            pl.BlockSpec((ROW_BLK, H), lambda i: (i, 0)),
            pl.BlockSpec((1, 1, ROW_BLK), lambda i: (i, 0, 0)),
            pl.BlockSpec((H, H // 2), lambda i: (0, 0)),
            pl.BlockSpec((1, H // 2), lambda i: (0, 0)),
            pl.BlockSpec((1, H // 2), lambda i: (0, 0)),
            pl.BlockSpec((1, 1), lambda i: (0, 0)),
            pl.BlockSpec((H, H), lambda i: (0, 0)),
            pl.BlockSpec((1, H), lambda i: (0, 0)),
        ],
        out_specs=pl.BlockSpec((NG, NG), lambda i: (0, 0)),
        out_shape=jax.ShapeDtypeStruct((NG, NG), jnp.float32),
        scratch_shapes=[pltpu.VMEM((NG, H + 1), jnp.float32)],
    )(x, bidx3, params['W_r1'], params['b_r1'].reshape(1, -1),
      params['W_r2'].reshape(1, -1), params['b_r2'].reshape(1, 1),
      params['W_last'], params['b_last'].reshape(1, -1))
    return res[:, 0]


def kernel(node_fts, automic_num, edge_fts, edge_index, F_norm_edge, F_dig,
           node_deg_vec, node_deg_mat, lap_mat, k_eig_val, k_eig_vec,
           num_nodes, norm_n, batch_idx, params):
    i32 = jnp.int32
    src = edge_index[0].astype(i32)
    dst = edge_index[1].astype(i32)
    perm = jnp.argsort(dst).astype(i32)
    dst_s = jnp.take(dst, perm)
    src_s = jnp.take(src, perm)
    pad = jnp.zeros((E_PAD - E,), i32)
    srcp = jnp.concatenate([src_s, pad])
    dstp = jnp.concatenate([dst_s, pad])
    permp = jnp.concatenate([perm, pad])
    bnd = jnp.searchsorted(
        dst_s, jnp.arange(NSEG + 1, dtype=i32) * RA).astype(i32)
    ebounds = jnp.concatenate([bnd, jnp.full((168 - NSEG - 1,), E, i32)])

    layers = params['layers']
    w1s = [lp['W_pre'][:H] for lp in layers]
    w2s = [lp['W_pre'][H:2 * H] for lp in layers]
    wc = jnp.concatenate(
        [params['W_edge'] @ lp['W_pre'][2 * H:] for lp in layers], axis=1)
    bc = jnp.concatenate(
        [(params['b_edge'] @ lp['W_pre'][2 * H:] + lp['b_pre']).reshape(1, H)
         for lp in layers], axis=1)
    emb_pad = jnp.zeros((16, 16), jnp.float32).at[:10].set(params['emb'])
    automic_f = automic_num.astype(jnp.float32).reshape(N, 1)
    efts_pad = jnp.pad(edge_fts, ((0, 0), (0, PC - 16)))

    efts_sp = _permute_k(efts_pad, permp)
    e3 = _edge_linear(efts_sp, wc, bc)

    x, t1, t2 = _encoder(
        node_fts, automic_f, params['W_node'],
        params['b_node'].reshape(1, -1), emb_pad, params['W_first'],
        params['b_first'].reshape(1, -1), w1s[0], w2s[0])

    for l, lp in enumerate(layers):
        s, q, x4, n4 = _edge_k(t1, t2, e3[l], srcp, dstp, ebounds)
        h, stats = _post1(s, q, x4, n4, x, norm_n, lp['W_post'],
                          lp['b_post'].reshape(1, -1))
        gamma = lp['gamma'].reshape(1, -1)
        beta = lp['beta'].reshape(1, -1)
        if l + 1 < len(layers):
            x, t1, t2 = _post2_next(
                h, stats, x, gamma, beta, w1s[l + 1], w2s[l + 1])
        else:
            (x,) = _post2_last(h, stats, x, gamma, beta)
    return _readout(x, batch_idx, params)
